# baseline (speedup 1.0000x reference)
import functools

import jax
import jax.numpy as jnp
from jax.experimental import pallas as pl
from jax.experimental.pallas import tpu as pltpu

_BLOCK = 1000


def _gc_block_kernel(item_ref, user_ref, r_ref, c_ref, Wu_ref, bu_ref,
                     Wv_ref, bv_ref, Wl_ref, bl_ref, u_out_ref, v_out_ref, *,
                     num_classes, hidden):
    x_item = item_ref[...]
    x_user = user_ref[...]
    rcol = r_ref[...]
    ccol = c_ref[...]
    zu = jnp.dot(x_item, Wu_ref[...], preferred_element_type=jnp.float32)
    zv = jnp.dot(x_user, Wv_ref[...], preferred_element_type=jnp.float32)
    H = hidden
    klane = jax.lax.broadcasted_iota(jnp.int32, (rcol.shape[0], num_classes), 1)
    m = jnp.where(rcol == klane, ccol, 0.0)
    un = jnp.dot(m, bu_ref[...], preferred_element_type=jnp.float32)
    vn = jnp.dot(m, bv_ref[...], preferred_element_type=jnp.float32)
    for cc in range(num_classes):
        mc = m[:, cc:cc + 1]
        un += mc * zu[:, cc * H:(cc + 1) * H]
        vn += mc * zv[:, cc * H:(cc + 1) * H]
    hu = jnp.maximum(un, 0.0)
    hv = jnp.maximum(vn, 0.0)
    ou = jnp.dot(hu, Wl_ref[...], preferred_element_type=jnp.float32) + bl_ref[...]
    ov = jnp.dot(hv, Wl_ref[...], preferred_element_type=jnp.float32) + bl_ref[...]
    u_out_ref[...] = jnp.maximum(ou, 0.0)
    v_out_ref[...] = jnp.maximum(ov, 0.0)


def kernel(user, item, r, c, Wu, bu, Wv, bv, Wl, bl):
    N, I = user.shape
    C, H, _ = Wu.shape
    O = Wl.shape[0]
    Wu_all = jnp.transpose(Wu, (2, 0, 1)).reshape(I, C * H)
    Wv_all = jnp.transpose(Wv, (2, 0, 1)).reshape(I, C * H)
    WlT = jnp.transpose(Wl)
    r2 = r.reshape(N, 1)
    c2 = c.reshape(N, 1)
    nb = N // _BLOCK
    bs_x = pl.BlockSpec((_BLOCK, I), lambda i: (i, 0))
    bs_i = pl.BlockSpec((_BLOCK, 1), lambda i: (i, 0))
    bs_W = pl.BlockSpec((I, C * H), lambda i: (0, 0))
    bs_b = pl.BlockSpec((C, H), lambda i: (0, 0))
    bs_Wl = pl.BlockSpec((H, O), lambda i: (0, 0))
    bs_bl = pl.BlockSpec((1, O), lambda i: (0, 0))
    bs_out = pl.BlockSpec((_BLOCK, O), lambda i: (i, 0))
    u_out, v_out = pl.pallas_call(
        functools.partial(_gc_block_kernel, num_classes=C, hidden=H),
        grid=(nb,),
        in_specs=[bs_x, bs_x, bs_i, bs_i, bs_W, bs_b, bs_W, bs_b, bs_Wl, bs_bl],
        out_specs=[bs_out, bs_out],
        out_shape=[jax.ShapeDtypeStruct((N, O), jnp.float32)] * 2,
        compiler_params=pltpu.CompilerParams(
            dimension_semantics=("parallel",)),
    )(item, user, r2, c2, Wu_all, bu, Wv_all, bv, WlT, bl.reshape(1, O))
    return (u_out, v_out)


# speedup vs baseline: 1.4142x; 1.4142x over previous
import functools

import jax
import jax.numpy as jnp
from jax.experimental import pallas as pl
from jax.experimental.pallas import tpu as pltpu

_BLOCK = 1000


def _gc_block_kernel(item_ref, user_ref, r_ref, c_ref, Wu_ref, bu_ref,
                     Wv_ref, bv_ref, Wl_ref, bl_ref, u_out_ref, v_out_ref,
                     WuT_s, WvT_s, WlT_s, *, num_classes, hidden):
    @pl.when(pl.program_id(0) == 0)
    def _init():
        WuT_s[...] = Wu_ref[...].T
        WvT_s[...] = Wv_ref[...].T
        WlT_s[...] = Wl_ref[...].T

    x_item = item_ref[...]
    x_user = user_ref[...]
    rl = r_ref[0]
    cl = c_ref[0]
    B = rl.shape[1]
    zu = jnp.dot(x_item, WuT_s[...], preferred_element_type=jnp.float32)
    zv = jnp.dot(x_user, WvT_s[...], preferred_element_type=jnp.float32)
    H = hidden
    ksub = jax.lax.broadcasted_iota(jnp.int32, (num_classes, B), 0)
    m = jnp.where(rl == ksub, cl, 0.0).T
    un = jnp.dot(m, bu_ref[...], preferred_element_type=jnp.float32)
    vn = jnp.dot(m, bv_ref[...], preferred_element_type=jnp.float32)
    for cc in range(num_classes):
        mc = m[:, cc:cc + 1]
        un += mc * zu[:, cc * H:(cc + 1) * H]
        vn += mc * zv[:, cc * H:(cc + 1) * H]
    hu = jnp.maximum(un, 0.0)
    hv = jnp.maximum(vn, 0.0)
    ou = jnp.dot(hu, WlT_s[...], preferred_element_type=jnp.float32) + bl_ref[...]
    ov = jnp.dot(hv, WlT_s[...], preferred_element_type=jnp.float32) + bl_ref[...]
    u_out_ref[...] = jnp.maximum(ou, 0.0)
    v_out_ref[...] = jnp.maximum(ov, 0.0)


def kernel(user, item, r, c, Wu, bu, Wv, bv, Wl, bl):
    N, I = user.shape
    C, H, _ = Wu.shape
    O = Wl.shape[0]
    nb = N // _BLOCK
    r3 = r.reshape(nb, 1, _BLOCK)
    c3 = c.reshape(nb, 1, _BLOCK)
    bs_x = pl.BlockSpec((_BLOCK, I), lambda i: (i, 0))
    bs_l = pl.BlockSpec((1, 1, _BLOCK), lambda i: (i, 0, 0))
    bs_W = pl.BlockSpec((C * H, I), lambda i: (0, 0))
    bs_b = pl.BlockSpec((C, H), lambda i: (0, 0))
    bs_Wl = pl.BlockSpec((O, H), lambda i: (0, 0))
    bs_bl = pl.BlockSpec((1, O), lambda i: (0, 0))
    bs_out = pl.BlockSpec((_BLOCK, O), lambda i: (i, 0))
    u_out, v_out = pl.pallas_call(
        functools.partial(_gc_block_kernel, num_classes=C, hidden=H),
        grid=(nb,),
        in_specs=[bs_x, bs_x, bs_l, bs_l, bs_W, bs_b, bs_W, bs_b, bs_Wl, bs_bl],
        out_specs=[bs_out, bs_out],
        out_shape=[jax.ShapeDtypeStruct((N, O), jnp.float32)] * 2,
        scratch_shapes=[
            pltpu.VMEM((I, C * H), jnp.float32),
            pltpu.VMEM((I, C * H), jnp.float32),
            pltpu.VMEM((H, O), jnp.float32),
        ],
        compiler_params=pltpu.CompilerParams(
            dimension_semantics=("arbitrary",)),
    )(item, user, r3, c3, Wu.reshape(C * H, I), bu, Wv.reshape(C * H, I),
      bv, Wl, bl.reshape(1, O))
    return (u_out, v_out)
